# baseline (speedup 1.0000x reference)
import functools

import jax
import jax.numpy as jnp
from jax import lax
from jax.experimental import pallas as pl
from jax.experimental.pallas import tpu as pltpu
from jax.experimental.pallas import tpu_sc as plsc

N1, N2, N3 = 50000, 10000, 2048
E1, E2 = 320000, 65536
D_IN, D_H, N_CLS = 128, 128, 64

NC, NS = 2, 16
NW = NC * NS
L = 16
CHUNK = 128

C1 = 2560
E1P = C1 * CHUNK
PAD1 = E1P - E1
C2 = E2 // CHUNK
KBUF = 1
WIN = 32

N1H = 51200
N2H = 10240
N3H = 2048
ACC1 = 10240

_mesh = plsc.VectorSubcoreMesh(core_axis_name="c", subcore_axis_name="s")


def _fill(ref, value):
    v = jnp.full((L,), value, jnp.float32)
    if len(ref.shape) == 1:
        def body(i, c):
            ref[pl.ds(i * L, L)] = v
            return c
        lax.fori_loop(0, ref.shape[0] // L, body, 0)
    else:
        def body(r, c):
            for k in range(ref.shape[1] // L):
                ref[r, pl.ds(k * L, L)] = v
            return c
        lax.fori_loop(0, ref.shape[0], body, 0)


def _hist_body(s1_ref, s2_ref, d1_ref, d2_ref,
               o_s1, o_s2, o_d1, o_d2,
               h_s1, h_s2, h_d1, h_d2, idx_v, ones_v, buf_v, sem):
    cid = lax.axis_index("c")
    sid = lax.axis_index("s")
    _fill(ones_v, 1.0)
    _fill(buf_v, 0.0)

    def _zero(hist):
        n = hist.shape[0] // NS
        for r in range(n // CHUNK):
            pltpu.sync_copy(buf_v.at[pl.ds(0, CHUNK)],
                            hist.at[pl.ds(sid * n + r * CHUNK, CHUNK)])

    def _count(edge2d, hist, nchunks):
        nph = idx_v.shape[0]

        def fire(j, carry):
            pltpu.sync_copy(ones_v, hist.at[idx_v.at[j]], add=True)
            return carry

        for ph in range(nchunks // nph):
            pltpu.sync_copy(edge2d.at[pl.ds(sid * nchunks + ph * nph, nph)],
                            idx_v)
            lax.fori_loop(0, nph, fire, 0)

    def _drain(hist, out):
        n = hist.shape[0] // NS
        nb = min(buf_v.shape[0], n)
        for r in range(n // nb):
            pltpu.sync_copy(hist.at[pl.ds(sid * n + r * nb, nb)],
                            buf_v.at[pl.ds(0, nb)])
            pltpu.sync_copy(buf_v.at[pl.ds(0, nb)],
                            out.at[pl.ds(sid * n + r * nb, nb)])

    @pl.when(cid == 0)
    def _():
        _zero(h_s1)
        _zero(h_s2)

    @pl.when(cid == 1)
    def _():
        _zero(h_d1)
        _zero(h_d2)

    plsc.subcore_barrier()

    @pl.when(cid == 0)
    def _():
        _count(s1_ref, h_s1, C1 // NS)
        _count(s2_ref, h_s2, C2 // NS)

    @pl.when(cid == 1)
    def _():
        _count(d1_ref, h_d1, C1 // NS)
        _count(d2_ref, h_d2, C2 // NS)

    plsc.subcore_barrier()

    @pl.when(cid == 0)
    def _():
        _drain(h_s1, o_s1)
        _drain(h_s2, o_s2)

    @pl.when(cid == 1)
    def _():
        _drain(h_d1, o_d1)
        _drain(h_d2, o_d2)


_hist_call = pl.kernel(
    _hist_body,
    out_type=[jax.ShapeDtypeStruct((N1H,), jnp.float32),
              jax.ShapeDtypeStruct((N2H,), jnp.float32),
              jax.ShapeDtypeStruct((N2H,), jnp.float32),
              jax.ShapeDtypeStruct((N3H,), jnp.float32)],
    mesh=_mesh,
    scratch_types=[pltpu.VMEM_SHARED((N1H,), jnp.float32),
                   pltpu.VMEM_SHARED((N2H,), jnp.float32),
                   pltpu.VMEM_SHARED((N2H,), jnp.float32),
                   pltpu.VMEM_SHARED((N3H,), jnp.float32),
                   pltpu.VMEM((32, CHUNK), jnp.int32),
                   pltpu.VMEM((CHUNK,), jnp.float32),
                   pltpu.VMEM((640,), jnp.float32),
                   pltpu.SemaphoreType.DMA],
)


def _agg_body(nchunks, nphase, nacc, feat_ref, src_ref, dst_ref,
              out_ref, acc, src_v, dst_v, *bufs):
    cid = lax.axis_index("c")
    sid = lax.axis_index("s")
    wid = sid * NC + cid
    rows = list(bufs[:KBUF])
    sems = list(bufs[KBUF:])
    r0 = rows[0]
    nph = nchunks // nphase

    _fill(r0, 0.0)
    rows_per_tile = nacc // NS
    for r in range(rows_per_tile // CHUNK):
        pltpu.sync_copy(
            r0, acc.at[pl.ds(sid * rows_per_tile + r * CHUNK, CHUNK)])
    plsc.subcore_barrier()

    def _gather(j, k):
        return pltpu.make_async_copy(feat_ref.at[src_v.at[j]],
                                     rows[k], sems[k])

    for ph in range(nphase):
        base = wid * nchunks + ph * nph
        pltpu.sync_copy(src_ref.at[pl.ds(base, nph)], src_v)
        pltpu.sync_copy(dst_ref.at[pl.ds(base, nph)], dst_v)

        for k in range(KBUF):
            _gather(k, k).start()

        nblk = nph // KBUF

        def body(blk, carry):
            for k in range(KBUF):
                j = blk * KBUF + k
                _gather(j, k).wait()
                pltpu.sync_copy(rows[k], acc.at[dst_v.at[j]], add=True)

                @pl.when(blk < nblk - 1)
                def _():
                    _gather(j + KBUF, k).start()
            return carry

        lax.fori_loop(0, nblk, body, 0)

    plsc.subcore_barrier()

    out_rows = nacc // NS
    for r in range(out_rows // CHUNK):
        pltpu.sync_copy(acc.at[pl.ds(sid * out_rows + r * CHUNK, CHUNK)], r0)
        pltpu.sync_copy(
            r0,
            out_ref.at[pl.ds(cid * nacc + sid * out_rows + r * CHUNK, CHUNK)])


def _make_agg(nchunks_per_worker, nphase, nacc, d):
    body = functools.partial(_agg_body, nchunks_per_worker, nphase, nacc)
    nph = nchunks_per_worker // nphase
    return pl.kernel(
        body,
        out_type=jax.ShapeDtypeStruct((NC * nacc, d), jnp.float32),
        mesh=_mesh,
        scratch_types=[pltpu.VMEM_SHARED((nacc, d), jnp.float32),
                       pltpu.VMEM((nph, CHUNK), jnp.int32),
                       pltpu.VMEM((nph, CHUNK), jnp.int32)]
                      + [pltpu.VMEM((CHUNK, d), jnp.float32)] * KBUF
                      + [pltpu.SemaphoreType.DMA] * KBUF,
    )


_agg1_call = _make_agg(C1 // NW, 2, ACC1, D_H)
_agg2_call = _make_agg(C2 // NW, 2, N3, D_H)


def _scale_body(x_ref, deg_ref, o_ref):
    norm = lax.rsqrt(jnp.maximum(deg_ref[...], 1.0))
    o_ref[...] = x_ref[...] * norm


def _mlp_body(a0_ref, a1_ref, di_ref, do_ref, w1_ref, b1_ref, w2_ref, p_ref):
    a = a0_ref[...] + a1_ref[...]
    h = jnp.dot(a, w1_ref[...], preferred_element_type=jnp.float32)
    h = h * lax.rsqrt(jnp.maximum(di_ref[...], 1.0)) + b1_ref[...]
    h = jnp.maximum(h, 0.0)
    h = h * lax.rsqrt(jnp.maximum(do_ref[...], 1.0))
    p_ref[...] = jnp.dot(h, w2_ref[...], preferred_element_type=jnp.float32)


def _final_body(e0_ref, e1_ref, deg_ref, b2_ref, o_ref):
    agg = e0_ref[:, :N_CLS] + e1_ref[:, :N_CLS]
    o_ref[...] = agg * lax.rsqrt(jnp.maximum(deg_ref[...], 1.0)) + b2_ref[...]


def kernel(in_feat, mfg1_src, mfg1_dst, mfg2_src, mfg2_dst, W1, b1, W2, b2):
    i32 = jnp.int32
    s1 = mfg1_src.astype(i32)
    d1 = mfg1_dst.astype(i32)
    s2 = mfg2_src.astype(i32)
    d2 = mfg2_dst.astype(i32)

    pad = jnp.arange(PAD1, dtype=i32)
    s1h = jnp.concatenate([s1, N1 + pad % 1024]).reshape(C1, CHUNK)
    s1g = jnp.concatenate([s1, pad % N1]).reshape(C1, CHUNK)
    d1p = jnp.concatenate([d1, N2 + pad % 224]).reshape(C1, CHUNK)
    s2r = s2.reshape(C2, CHUNK)
    d2r = d2.reshape(C2, CHUNK)

    h_s1, h_s2, h_d1, h_d2 = _hist_call(s1h, s2r, d1p, d2r)
    deg1o = h_s1[:N1].reshape(N1, 1)
    deg2o = h_s2[:N2].reshape(N2, 1)
    deg1i = h_d1[:N2].reshape(N2, 1)
    deg2i = h_d2[:N3].reshape(N3, 1)

    blk = 1000
    feat_scaled = pl.pallas_call(
        _scale_body,
        grid=(N1 // blk,),
        in_specs=[pl.BlockSpec((blk, D_IN), lambda i: (i, 0)),
                  pl.BlockSpec((blk, 1), lambda i: (i, 0))],
        out_specs=pl.BlockSpec((blk, D_IN), lambda i: (i, 0)),
        out_shape=jax.ShapeDtypeStruct((N1, D_IN), jnp.float32),
        compiler_params=pltpu.CompilerParams(
            dimension_semantics=("parallel",)),
    )(in_feat, deg1o)

    agg1 = _agg1_call(feat_scaled, s1g, d1p)

    p = pl.pallas_call(
        _mlp_body,
        grid=(N2 // blk,),
        in_specs=[pl.BlockSpec((blk, D_H), lambda i: (i, 0)),
                  pl.BlockSpec((blk, D_H), lambda i: (i, 0)),
                  pl.BlockSpec((blk, 1), lambda i: (i, 0)),
                  pl.BlockSpec((blk, 1), lambda i: (i, 0)),
                  pl.BlockSpec((D_H, D_H), lambda i: (0, 0)),
                  pl.BlockSpec((1, D_H), lambda i: (0, 0)),
                  pl.BlockSpec((D_H, D_H), lambda i: (0, 0))],
        out_specs=pl.BlockSpec((blk, D_H), lambda i: (i, 0)),
        out_shape=jax.ShapeDtypeStruct((N2, D_H), jnp.float32),
        compiler_params=pltpu.CompilerParams(
            dimension_semantics=("parallel",)),
    )(agg1[:N2], agg1[ACC1:ACC1 + N2], deg1i, deg2o, W1,
      b1.reshape(1, D_H), jnp.pad(W2, ((0, 0), (0, D_H - N_CLS))))

    agg2 = _agg2_call(p, s2r, d2r)

    out = pl.pallas_call(
        _final_body,
        in_specs=[pl.BlockSpec((N3, D_H), lambda: (0, 0)),
                  pl.BlockSpec((N3, D_H), lambda: (0, 0)),
                  pl.BlockSpec((N3, 1), lambda: (0, 0)),
                  pl.BlockSpec((1, N_CLS), lambda: (0, 0))],
        out_specs=pl.BlockSpec((N3, N_CLS), lambda: (0, 0)),
        out_shape=jax.ShapeDtypeStruct((N3, N_CLS), jnp.float32),
    )(agg2[:N3], agg2[N3:2 * N3], deg2i, b2.reshape(1, N_CLS))

    return out


# speedup vs baseline: 1.0834x; 1.0834x over previous
import functools

import jax
import jax.numpy as jnp
from jax import lax
from jax.experimental import pallas as pl
from jax.experimental.pallas import tpu as pltpu
from jax.experimental.pallas import tpu_sc as plsc

N1, N2, N3 = 50000, 10000, 2048
E1, E2 = 320000, 65536
D_IN, D_H, N_CLS = 128, 128, 64

NC, NS = 2, 16
NW = NC * NS
L = 16
CHUNK = 128

C1 = 2560
E1P = C1 * CHUNK
PAD1 = E1P - E1
C2 = E2 // CHUNK
KBUF = 2

N1H = 51200
N2H = 10240
N3H = 2048
ACC1 = 10240

_mesh = plsc.VectorSubcoreMesh(core_axis_name="c", subcore_axis_name="s")


def _fill(ref, value):
    v = jnp.full((L,), value, jnp.float32)
    if len(ref.shape) == 1:
        def body(i, c):
            ref[pl.ds(i * L, L)] = v
            return c
        lax.fori_loop(0, ref.shape[0] // L, body, 0)
    else:
        def body(r, c):
            for k in range(ref.shape[1] // L):
                ref[r, pl.ds(k * L, L)] = v
            return c
        lax.fori_loop(0, ref.shape[0], body, 0)


def _count_chunks(edge2d, hist, idx_v, ones_v, wid, nchunks):
    nph = min(idx_v.shape[0], nchunks)

    def fire(j, carry):
        pltpu.sync_copy(ones_v, hist.at[idx_v.at[j]], add=True)
        return carry

    for ph in range(nchunks // nph):
        pltpu.sync_copy(
            edge2d.at[pl.ds(wid * nchunks + ph * nph, nph)],
            idx_v.at[pl.ds(0, nph)])
        lax.fori_loop(0, nph, fire, 0)


def _drain_hist(hist, out, buf128, sid, cid):
    n = hist.shape[0] // NS
    for r in range(n // CHUNK):
        pltpu.sync_copy(hist.at[pl.ds(sid * n + r * CHUNK, CHUNK)], buf128)
        pltpu.sync_copy(
            buf128,
            out.at[pl.ds(cid * hist.shape[0] + sid * n + r * CHUNK, CHUNK)])


def _hist1_body(s1_ref, o_s1, h_s1, idx_v, ones_v):
    cid = lax.axis_index("c")
    sid = lax.axis_index("s")
    wid = sid * NC + cid

    n = N1H // NS
    _fill(ones_v, 0.0)
    for r in range(n // CHUNK):
        pltpu.sync_copy(ones_v, h_s1.at[pl.ds(sid * n + r * CHUNK, CHUNK)])
    plsc.subcore_barrier()

    _fill(ones_v, 1.0)
    _count_chunks(s1_ref, h_s1, idx_v, ones_v, wid, C1 // NW)
    plsc.subcore_barrier()

    for r in range(n // CHUNK):
        pltpu.sync_copy(h_s1.at[pl.ds(sid * n + r * CHUNK, CHUNK)], ones_v)
        pltpu.sync_copy(
            ones_v, o_s1.at[pl.ds(cid * N1H + sid * n + r * CHUNK, CHUNK)])


_hist1_call = pl.kernel(
    _hist1_body,
    out_type=jax.ShapeDtypeStruct((NC * N1H,), jnp.float32),
    mesh=_mesh,
    scratch_types=[pltpu.VMEM_SHARED((N1H,), jnp.float32),
                   pltpu.VMEM((40, CHUNK), jnp.int32),
                   pltpu.VMEM((CHUNK,), jnp.float32)],
)


def _agg1_body(feat_ref, src_ref, dst_ref, s2_ref, d2_ref,
               out_ref, o_d1, o_s2, o_d2,
               acc, h_d1, h_s2, h_d2, src_v, dst_v, r0, r1, s0, s1):
    cid = lax.axis_index("c")
    sid = lax.axis_index("s")
    wid = sid * NC + cid
    rows = [r0, r1]
    sems = [s0, s1]
    nchunks = C1 // NW
    nph = nchunks // 2

    _fill(r0, 0.0)
    rows_per_tile = ACC1 // NS
    for r in range(rows_per_tile // CHUNK):
        pltpu.sync_copy(
            r0, acc.at[pl.ds(sid * rows_per_tile + r * CHUNK, CHUNK)])
    for hist in (h_d1, h_s2, h_d2):
        n = hist.shape[0] // NS
        for r in range(n // CHUNK):
            pltpu.sync_copy(r0.at[0],
                            hist.at[pl.ds(sid * n + r * CHUNK, CHUNK)])
    plsc.subcore_barrier()

    def _gather(j, k):
        return pltpu.make_async_copy(feat_ref.at[src_v.at[j]],
                                     rows[k], sems[k])

    for ph in range(2):
        base = wid * nchunks + ph * nph
        pltpu.sync_copy(src_ref.at[pl.ds(base, nph)], src_v)
        pltpu.sync_copy(dst_ref.at[pl.ds(base, nph)], dst_v)

        for k in range(KBUF):
            _gather(k, k).start()

        nblk = nph // KBUF

        def body(blk, carry):
            for k in range(KBUF):
                j = blk * KBUF + k
                _gather(j, k).wait()
                pltpu.sync_copy(rows[k], acc.at[dst_v.at[j]], add=True)

                @pl.when(blk < nblk - 1)
                def _():
                    _gather(j + KBUF, k).start()
            return carry

        lax.fori_loop(0, nblk, body, 0)

    ones_v = r1.at[0]
    _fill(r1, 1.0)
    _count_chunks(dst_ref, h_d1, src_v, ones_v, wid, C1 // NW)
    _count_chunks(s2_ref, h_s2, src_v, ones_v, wid, C2 // NW)
    _count_chunks(d2_ref, h_d2, src_v, ones_v, wid, C2 // NW)
    plsc.subcore_barrier()

    out_rows = ACC1 // NS
    for r in range(out_rows // CHUNK):
        pltpu.sync_copy(acc.at[pl.ds(sid * out_rows + r * CHUNK, CHUNK)], r0)
        pltpu.sync_copy(
            r0,
            out_ref.at[pl.ds(cid * ACC1 + sid * out_rows + r * CHUNK, CHUNK)])
    buf128 = r1.at[1]
    _drain_hist(h_d1, o_d1, buf128, sid, cid)
    _drain_hist(h_s2, o_s2, buf128, sid, cid)
    _drain_hist(h_d2, o_d2, buf128, sid, cid)


_agg1_call = pl.kernel(
    _agg1_body,
    out_type=[jax.ShapeDtypeStruct((NC * ACC1, D_H), jnp.float32),
              jax.ShapeDtypeStruct((NC * N2H,), jnp.float32),
              jax.ShapeDtypeStruct((NC * N2H,), jnp.float32),
              jax.ShapeDtypeStruct((NC * N3H,), jnp.float32)],
    mesh=_mesh,
    scratch_types=[pltpu.VMEM_SHARED((ACC1, D_H), jnp.float32),
                   pltpu.VMEM_SHARED((N2H,), jnp.float32),
                   pltpu.VMEM_SHARED((N2H,), jnp.float32),
                   pltpu.VMEM_SHARED((N3H,), jnp.float32),
                   pltpu.VMEM((40, CHUNK), jnp.int32),
                   pltpu.VMEM((40, CHUNK), jnp.int32),
                   pltpu.VMEM((CHUNK, D_H), jnp.float32),
                   pltpu.VMEM((CHUNK, D_H), jnp.float32),
                   pltpu.SemaphoreType.DMA,
                   pltpu.SemaphoreType.DMA],
)


def _agg2_body(feat_ref, src_ref, dst_ref, out_ref,
               acc, src_v, dst_v, r0, r1, s0, s1):
    cid = lax.axis_index("c")
    sid = lax.axis_index("s")
    wid = sid * NC + cid
    rows = [r0, r1]
    sems = [s0, s1]
    nchunks = C2 // NW

    _fill(r0, 0.0)
    rows_per_tile = N3 // NS
    pltpu.sync_copy(r0, acc.at[pl.ds(sid * rows_per_tile, rows_per_tile)])
    plsc.subcore_barrier()

    base = wid * nchunks
    pltpu.sync_copy(src_ref.at[pl.ds(base, nchunks)], src_v)
    pltpu.sync_copy(dst_ref.at[pl.ds(base, nchunks)], dst_v)

    def _gather(j, k):
        return pltpu.make_async_copy(feat_ref.at[src_v.at[j]],
                                     rows[k], sems[k])

    for k in range(KBUF):
        _gather(k, k).start()

    nblk = nchunks // KBUF

    def body(blk, carry):
        for k in range(KBUF):
            j = blk * KBUF + k
            _gather(j, k).wait()
            pltpu.sync_copy(rows[k], acc.at[dst_v.at[j]], add=True)

            @pl.when(blk < nblk - 1)
            def _():
                _gather(j + KBUF, k).start()
        return carry

    lax.fori_loop(0, nblk, body, 0)
    plsc.subcore_barrier()

    out_rows = N3 // NS
    pltpu.sync_copy(acc.at[pl.ds(sid * out_rows, out_rows)], r0)
    pltpu.sync_copy(r0, out_ref.at[pl.ds(cid * N3 + sid * out_rows,
                                         out_rows)])


_agg2_call = pl.kernel(
    _agg2_body,
    out_type=jax.ShapeDtypeStruct((NC * N3, D_H), jnp.float32),
    mesh=_mesh,
    scratch_types=[pltpu.VMEM_SHARED((N3, D_H), jnp.float32),
                   pltpu.VMEM((16, CHUNK), jnp.int32),
                   pltpu.VMEM((16, CHUNK), jnp.int32),
                   pltpu.VMEM((CHUNK, D_H), jnp.float32),
                   pltpu.VMEM((CHUNK, D_H), jnp.float32),
                   pltpu.SemaphoreType.DMA,
                   pltpu.SemaphoreType.DMA],
)


def _scale_body(x_ref, d0_ref, d1_ref, o_ref):
    deg = d0_ref[...] + d1_ref[...]
    o_ref[...] = x_ref[...] * lax.rsqrt(jnp.maximum(deg, 1.0))


def _mlp_body(a0_ref, a1_ref, di0_ref, di1_ref, do0_ref, do1_ref,
              w1_ref, b1_ref, w2_ref, p_ref):
    a = a0_ref[...] + a1_ref[...]
    h = jnp.dot(a, w1_ref[...], preferred_element_type=jnp.float32)
    di = di0_ref[...] + di1_ref[...]
    h = h * lax.rsqrt(jnp.maximum(di, 1.0)) + b1_ref[...]
    h = jnp.maximum(h, 0.0)
    do = do0_ref[...] + do1_ref[...]
    h = h * lax.rsqrt(jnp.maximum(do, 1.0))
    p_ref[...] = jnp.dot(h, w2_ref[...], preferred_element_type=jnp.float32)


def _final_body(e0_ref, e1_ref, d0_ref, d1_ref, b2_ref, o_ref):
    agg = e0_ref[:, :N_CLS] + e1_ref[:, :N_CLS]
    deg = d0_ref[...] + d1_ref[...]
    o_ref[...] = agg * lax.rsqrt(jnp.maximum(deg, 1.0)) + b2_ref[...]


def kernel(in_feat, mfg1_src, mfg1_dst, mfg2_src, mfg2_dst, W1, b1, W2, b2):
    i32 = jnp.int32
    s1 = mfg1_src.astype(i32)
    d1 = mfg1_dst.astype(i32)
    s2 = mfg2_src.astype(i32)
    d2 = mfg2_dst.astype(i32)

    pad = jnp.arange(PAD1, dtype=i32)
    s1h = jnp.concatenate([s1, N1 + pad % 1024]).reshape(C1, CHUNK)
    s1g = jnp.concatenate([s1, pad % N1]).reshape(C1, CHUNK)
    d1p = jnp.concatenate([d1, N2 + pad % 224]).reshape(C1, CHUNK)
    s2r = s2.reshape(C2, CHUNK)
    d2r = d2.reshape(C2, CHUNK)

    h_s1 = _hist1_call(s1h)
    d1o_a = h_s1[:N1].reshape(N1, 1)
    d1o_b = h_s1[N1H:N1H + N1].reshape(N1, 1)

    blk = 1000
    feat_scaled = pl.pallas_call(
        _scale_body,
        grid=(N1 // blk,),
        in_specs=[pl.BlockSpec((blk, D_IN), lambda i: (i, 0)),
                  pl.BlockSpec((blk, 1), lambda i: (i, 0)),
                  pl.BlockSpec((blk, 1), lambda i: (i, 0))],
        out_specs=pl.BlockSpec((blk, D_IN), lambda i: (i, 0)),
        out_shape=jax.ShapeDtypeStruct((N1, D_IN), jnp.float32),
        compiler_params=pltpu.CompilerParams(
            dimension_semantics=("parallel",)),
    )(in_feat, d1o_a, d1o_b)

    agg1, o_d1, o_s2, o_d2 = _agg1_call(feat_scaled, s1g, d1p, s2r, d2r)

    p = pl.pallas_call(
        _mlp_body,
        grid=(N2 // blk,),
        in_specs=[pl.BlockSpec((blk, D_H), lambda i: (i, 0)),
                  pl.BlockSpec((blk, D_H), lambda i: (i, 0)),
                  pl.BlockSpec((blk, 1), lambda i: (i, 0)),
                  pl.BlockSpec((blk, 1), lambda i: (i, 0)),
                  pl.BlockSpec((blk, 1), lambda i: (i, 0)),
                  pl.BlockSpec((blk, 1), lambda i: (i, 0)),
                  pl.BlockSpec((D_H, D_H), lambda i: (0, 0)),
                  pl.BlockSpec((1, D_H), lambda i: (0, 0)),
                  pl.BlockSpec((D_H, D_H), lambda i: (0, 0))],
        out_specs=pl.BlockSpec((blk, D_H), lambda i: (i, 0)),
        out_shape=jax.ShapeDtypeStruct((N2, D_H), jnp.float32),
        compiler_params=pltpu.CompilerParams(
            dimension_semantics=("parallel",)),
    )(agg1[:N2], agg1[ACC1:ACC1 + N2],
      o_d1[:N2].reshape(N2, 1), o_d1[N2H:N2H + N2].reshape(N2, 1),
      o_s2[:N2].reshape(N2, 1), o_s2[N2H:N2H + N2].reshape(N2, 1),
      W1, b1.reshape(1, D_H), jnp.pad(W2, ((0, 0), (0, D_H - N_CLS))))

    agg2 = _agg2_call(p, s2r, d2r)

    out = pl.pallas_call(
        _final_body,
        in_specs=[pl.BlockSpec((N3, D_H), lambda: (0, 0)),
                  pl.BlockSpec((N3, D_H), lambda: (0, 0)),
                  pl.BlockSpec((N3, 1), lambda: (0, 0)),
                  pl.BlockSpec((N3, 1), lambda: (0, 0)),
                  pl.BlockSpec((1, N_CLS), lambda: (0, 0))],
        out_specs=pl.BlockSpec((N3, N_CLS), lambda: (0, 0)),
        out_shape=jax.ShapeDtypeStruct((N3, N_CLS), jnp.float32),
    )(agg2[:N3], agg2[N3:2 * N3],
      o_d2[:N3].reshape(N3, 1), o_d2[N3H:N3H + N3].reshape(N3, 1),
      b2.reshape(1, N_CLS))

    return out


# speedup vs baseline: 1.2766x; 1.1784x over previous
import functools

import jax
import jax.numpy as jnp
from jax import lax
from jax.experimental import pallas as pl
from jax.experimental.pallas import tpu as pltpu
from jax.experimental.pallas import tpu_sc as plsc

N1, N2, N3 = 50000, 10000, 2048
E1, E2 = 320000, 65536
D_IN, D_H, N_CLS = 128, 128, 64

NC, NS = 2, 16
NW = NC * NS
L = 16
CHUNK = 128

C1 = 2560
E1P = C1 * CHUNK
PAD1 = E1P - E1
C2 = E2 // CHUNK

N1H = 51200
N2H = 10240
N3H = 2048
ACC1 = 10240

_mesh = plsc.VectorSubcoreMesh(core_axis_name="c", subcore_axis_name="s")


def _fill(ref, value):
    v = jnp.full((L,), value, jnp.float32)
    if len(ref.shape) == 1:
        def body(i, c):
            ref[pl.ds(i * L, L)] = v
            return c
        lax.fori_loop(0, ref.shape[0] // L, body, 0)
    else:
        def body(r, c):
            for k in range(ref.shape[1] // L):
                ref[r, pl.ds(k * L, L)] = v
            return c
        lax.fori_loop(0, ref.shape[0], body, 0)


def _hist_body(s1_ref, s2_ref, d1_ref, d2_ref,
               o_s1, o_s2, o_d1, o_d2,
               h_s1, h_s2, h_d1, h_d2, idx_v, ones_v, buf_v, sem):
    cid = lax.axis_index("c")
    sid = lax.axis_index("s")
    _fill(ones_v, 1.0)
    _fill(buf_v, 0.0)

    def _zero(hist):
        n = hist.shape[0] // NS
        for r in range(n // CHUNK):
            pltpu.sync_copy(buf_v.at[pl.ds(0, CHUNK)],
                            hist.at[pl.ds(sid * n + r * CHUNK, CHUNK)])

    def _count(edge2d, hist, nchunks):
        nph = idx_v.shape[0]

        def fire(j, carry):
            pltpu.sync_copy(ones_v, hist.at[idx_v.at[j]], add=True)
            return carry

        for ph in range(nchunks // nph):
            pltpu.sync_copy(edge2d.at[pl.ds(sid * nchunks + ph * nph, nph)],
                            idx_v)
            lax.fori_loop(0, nph, fire, 0)

    def _drain(hist, out):
        n = hist.shape[0] // NS
        nb = min(buf_v.shape[0], n)
        for r in range(n // nb):
            pltpu.sync_copy(hist.at[pl.ds(sid * n + r * nb, nb)],
                            buf_v.at[pl.ds(0, nb)])
            pltpu.sync_copy(buf_v.at[pl.ds(0, nb)],
                            out.at[pl.ds(sid * n + r * nb, nb)])

    @pl.when(cid == 0)
    def _():
        _zero(h_s1)
        _zero(h_s2)

    @pl.when(cid == 1)
    def _():
        _zero(h_d1)
        _zero(h_d2)

    plsc.subcore_barrier()

    @pl.when(cid == 0)
    def _():
        _count(s1_ref, h_s1, C1 // NS)
        _count(s2_ref, h_s2, C2 // NS)

    @pl.when(cid == 1)
    def _():
        _count(d1_ref, h_d1, C1 // NS)
        _count(d2_ref, h_d2, C2 // NS)

    plsc.subcore_barrier()

    @pl.when(cid == 0)
    def _():
        _drain(h_s1, o_s1)
        _drain(h_s2, o_s2)

    @pl.when(cid == 1)
    def _():
        _drain(h_d1, o_d1)
        _drain(h_d2, o_d2)


_hist_call = pl.kernel(
    _hist_body,
    out_type=[jax.ShapeDtypeStruct((N1H,), jnp.float32),
              jax.ShapeDtypeStruct((N2H,), jnp.float32),
              jax.ShapeDtypeStruct((N2H,), jnp.float32),
              jax.ShapeDtypeStruct((N3H,), jnp.float32)],
    mesh=_mesh,
    scratch_types=[pltpu.VMEM_SHARED((N1H,), jnp.float32),
                   pltpu.VMEM_SHARED((N2H,), jnp.float32),
                   pltpu.VMEM_SHARED((N2H,), jnp.float32),
                   pltpu.VMEM_SHARED((N3H,), jnp.float32),
                   pltpu.VMEM((32, CHUNK), jnp.int32),
                   pltpu.VMEM((CHUNK,), jnp.float32),
                   pltpu.VMEM((640,), jnp.float32),
                   pltpu.SemaphoreType.DMA],
)


def _agg_body(nchunks, nphase, nacc, kbuf, feat_ref, src_ref, dst_ref,
              out_ref, acc, src_v, dst_v, *bufs):
    cid = lax.axis_index("c")
    sid = lax.axis_index("s")
    wid = sid * NC + cid
    rows = list(bufs[:kbuf])
    sems = list(bufs[kbuf:])
    r0 = rows[0]
    nph = nchunks // nphase

    _fill(r0, 0.0)
    rows_per_tile = nacc // NS
    for r in range(rows_per_tile // CHUNK):
        pltpu.sync_copy(
            r0, acc.at[pl.ds(sid * rows_per_tile + r * CHUNK, CHUNK)])
    plsc.subcore_barrier()

    def _gather(j, k):
        return pltpu.make_async_copy(feat_ref.at[src_v.at[j]],
                                     rows[k], sems[k])

    for ph in range(nphase):
        base = wid * nchunks + ph * nph
        pltpu.sync_copy(src_ref.at[pl.ds(base, nph)], src_v)
        pltpu.sync_copy(dst_ref.at[pl.ds(base, nph)], dst_v)

        for k in range(kbuf):
            _gather(k, k).start()

        nblk = nph // kbuf

        def body(blk, carry):
            for k in range(kbuf):
                j = blk * kbuf + k
                _gather(j, k).wait()
                pltpu.sync_copy(rows[k], acc.at[dst_v.at[j]], add=True)

                @pl.when(blk < nblk - 1)
                def _():
                    _gather(j + kbuf, k).start()
            return carry

        lax.fori_loop(0, nblk, body, 0)

    plsc.subcore_barrier()

    out_rows = nacc // NS
    for r in range(out_rows // CHUNK):
        pltpu.sync_copy(acc.at[pl.ds(sid * out_rows + r * CHUNK, CHUNK)], r0)
        pltpu.sync_copy(
            r0,
            out_ref.at[pl.ds(cid * nacc + sid * out_rows + r * CHUNK, CHUNK)])


def _make_agg(nchunks_per_worker, nphase, nacc, d, kbuf):
    body = functools.partial(_agg_body, nchunks_per_worker, nphase, nacc,
                             kbuf)
    nph = nchunks_per_worker // nphase
    return pl.kernel(
        body,
        out_type=jax.ShapeDtypeStruct((NC * nacc, d), jnp.float32),
        mesh=_mesh,
        scratch_types=[pltpu.VMEM_SHARED((nacc, d), jnp.float32),
                       pltpu.VMEM((nph, CHUNK), jnp.int32),
                       pltpu.VMEM((nph, CHUNK), jnp.int32)]
                      + [pltpu.VMEM((CHUNK, d), jnp.float32)] * kbuf
                      + [pltpu.SemaphoreType.DMA] * kbuf,
    )


_agg1_call = _make_agg(C1 // NW, 2, ACC1, D_H, 2)
_agg2_call = _make_agg(C2 // NW, 2, N3, D_H, 4)


def _scale_body(x_ref, deg_ref, o_ref):
    norm = lax.rsqrt(jnp.maximum(deg_ref[...], 1.0))
    o_ref[...] = x_ref[...] * norm


def _mlp_body(a0_ref, a1_ref, di_ref, do_ref, w1_ref, b1_ref, w2_ref, p_ref):
    a = a0_ref[...] + a1_ref[...]
    h = jnp.dot(a, w1_ref[...], preferred_element_type=jnp.float32)
    h = h * lax.rsqrt(jnp.maximum(di_ref[...], 1.0)) + b1_ref[...]
    h = jnp.maximum(h, 0.0)
    h = h * lax.rsqrt(jnp.maximum(do_ref[...], 1.0))
    p_ref[...] = jnp.dot(h, w2_ref[...], preferred_element_type=jnp.float32)


def _final_body(e0_ref, e1_ref, deg_ref, b2_ref, o_ref):
    agg = e0_ref[:, :N_CLS] + e1_ref[:, :N_CLS]
    o_ref[...] = agg * lax.rsqrt(jnp.maximum(deg_ref[...], 1.0)) + b2_ref[...]


def kernel(in_feat, mfg1_src, mfg1_dst, mfg2_src, mfg2_dst, W1, b1, W2, b2):
    i32 = jnp.int32
    s1 = mfg1_src.astype(i32)
    d1 = mfg1_dst.astype(i32)
    s2 = mfg2_src.astype(i32)
    d2 = mfg2_dst.astype(i32)

    pad = jnp.arange(PAD1, dtype=i32)
    s1h = jnp.concatenate([s1, N1 + pad % 1024]).reshape(C1, CHUNK)
    s1g = jnp.concatenate([s1, pad % N1]).reshape(C1, CHUNK)
    d1p = jnp.concatenate([d1, N2 + pad % 224]).reshape(C1, CHUNK)
    s2r = s2.reshape(C2, CHUNK)
    d2r = d2.reshape(C2, CHUNK)

    h_s1, h_s2, h_d1, h_d2 = _hist_call(s1h, s2r, d1p, d2r)
    deg1o = h_s1[:N1].reshape(N1, 1)
    deg2o = h_s2[:N2].reshape(N2, 1)
    deg1i = h_d1[:N2].reshape(N2, 1)
    deg2i = h_d2[:N3].reshape(N3, 1)

    blk = 1000
    feat_scaled = pl.pallas_call(
        _scale_body,
        grid=(N1 // blk,),
        in_specs=[pl.BlockSpec((blk, D_IN), lambda i: (i, 0)),
                  pl.BlockSpec((blk, 1), lambda i: (i, 0))],
        out_specs=pl.BlockSpec((blk, D_IN), lambda i: (i, 0)),
        out_shape=jax.ShapeDtypeStruct((N1, D_IN), jnp.float32),
        compiler_params=pltpu.CompilerParams(
            dimension_semantics=("parallel",)),
    )(in_feat, deg1o)

    agg1 = _agg1_call(feat_scaled, s1g, d1p)

    p = pl.pallas_call(
        _mlp_body,
        grid=(N2 // blk,),
        in_specs=[pl.BlockSpec((blk, D_H), lambda i: (i, 0)),
                  pl.BlockSpec((blk, D_H), lambda i: (i, 0)),
                  pl.BlockSpec((blk, 1), lambda i: (i, 0)),
                  pl.BlockSpec((blk, 1), lambda i: (i, 0)),
                  pl.BlockSpec((D_H, D_H), lambda i: (0, 0)),
                  pl.BlockSpec((1, D_H), lambda i: (0, 0)),
                  pl.BlockSpec((D_H, D_H), lambda i: (0, 0))],
        out_specs=pl.BlockSpec((blk, D_H), lambda i: (i, 0)),
        out_shape=jax.ShapeDtypeStruct((N2, D_H), jnp.float32),
        compiler_params=pltpu.CompilerParams(
            dimension_semantics=("parallel",)),
    )(agg1[:N2], agg1[ACC1:ACC1 + N2], deg1i, deg2o, W1,
      b1.reshape(1, D_H), jnp.pad(W2, ((0, 0), (0, D_H - N_CLS))))

    agg2 = _agg2_call(p, s2r, d2r)

    out = pl.pallas_call(
        _final_body,
        in_specs=[pl.BlockSpec((N3, D_H), lambda: (0, 0)),
                  pl.BlockSpec((N3, D_H), lambda: (0, 0)),
                  pl.BlockSpec((N3, 1), lambda: (0, 0)),
                  pl.BlockSpec((1, N_CLS), lambda: (0, 0))],
        out_specs=pl.BlockSpec((N3, N_CLS), lambda: (0, 0)),
        out_shape=jax.ShapeDtypeStruct((N3, N_CLS), jnp.float32),
    )(agg2[:N3], agg2[N3:2 * N3], deg2i, b2.reshape(1, N_CLS))

    return out
